# Initial kernel scaffold; baseline (speedup 1.0000x reference)
import jax
import jax.numpy as jnp
from jax.experimental import pallas as pl


def kernel(pred, target):
    raise NotImplementedError("write your pallas kernel here")


# speedup vs baseline: 17.2109x; 17.2109x over previous
import jax
import jax.numpy as jnp
from jax import lax
from jax.experimental import pallas as pl
from jax.experimental.pallas import tpu as pltpu

_THRESH = 0.7
_MIN_KEPT = 100000
_IGNORE = 255

_HB = 32
_SUB = 8
_N_ITERS = 45


def _ohem_kernel(pred_ref, tgt_ref, out_ref, loss_buf, acc_ref, nblocks, total_rows):
    i = pl.program_id(0)
    C = pred_ref.shape[1]
    W = pred_ref.shape[3]

    @pl.when(i == 0)
    def _init():
        acc_ref[...] = jnp.zeros_like(acc_ref)

    for r in range(_HB // _SUB):
        rows = pl.ds(r * _SUB, _SUB)
        t = tgt_ref[0, rows, :]
        m = pred_ref[0, 0, rows, :]
        for c in range(1, C):
            m = jnp.maximum(m, pred_ref[0, c, rows, :])
        s = jnp.zeros((_SUB, W), jnp.float32)
        xt = jnp.zeros((_SUB, W), jnp.float32)
        for c in range(C):
            x = pred_ref[0, c, rows, :]
            s = s + jnp.exp(x - m)
            xt = xt + jnp.where(t == c, x, 0.0)
        valid = t != _IGNORE
        loss = m + jnp.log(s) - xt
        loss = jnp.where(valid, loss, 0.0)
        loss_buf[pl.ds(i * _HB + r * _SUB, _SUB), :] = loss
        acc_ref[0] += jnp.where(valid, 1.0, 0.0)
        acc_ref[1] += loss
        acc_ref[2] = jnp.maximum(acc_ref[2], loss)

    @pl.when(i == nblocks - 1)
    def _finish():
        cnt_valid = jnp.sum(acc_ref[0])
        tot_sum = jnp.sum(acc_ref[1])
        max_loss = jnp.max(acc_ref[2])

        keep = jnp.maximum(
            jnp.float32(_MIN_KEPT), jnp.floor(cnt_valid * jnp.float32(_THRESH))
        )
        keep = jnp.minimum(keep, cnt_valid)

        nchunks = total_rows // _HB

        def count_ge(t):
            def body(j, acc):
                chunk = loss_buf[pl.ds(j * _HB, _HB), :]
                ge = jnp.where(chunk >= t, 1.0, 0.0)
                part = ge[0:_SUB]
                for k in range(1, _HB // _SUB):
                    part = part + ge[k * _SUB:(k + 1) * _SUB]
                return acc + part
            vec = lax.fori_loop(0, nchunks, body, jnp.zeros((_SUB, W), jnp.float32))
            return jnp.sum(vec)

        def bs_body(_, carry):
            tl, th = carry
            tm = 0.5 * (tl + th)
            cm = count_ge(tm)
            take_lo = cm >= keep + 1.0
            return (jnp.where(take_lo, tm, tl), jnp.where(take_lo, th, tm))

        tl0 = jnp.float32(0.0)
        th0 = max_loss * jnp.float32(1.0001) + jnp.float32(1e-6)
        tl, th = lax.fori_loop(0, _N_ITERS, bs_body, (tl0, th0))

        def final_body(j, carry):
            acc_s, acc_c = carry
            chunk = loss_buf[pl.ds(j * _HB, _HB), :]
            ge = jnp.where(chunk >= th, 1.0, 0.0)
            kept = ge * chunk
            ps = kept[0:_SUB]
            pc = ge[0:_SUB]
            for k in range(1, _HB // _SUB):
                ps = ps + kept[k * _SUB:(k + 1) * _SUB]
                pc = pc + ge[k * _SUB:(k + 1) * _SUB]
            return (acc_s + ps, acc_c + pc)

        zero = jnp.zeros((_SUB, W), jnp.float32)
        acc_s, acc_c = lax.fori_loop(0, nchunks, final_body, (zero, zero))
        sum_hi = jnp.sum(acc_s)
        cnt_hi = jnp.sum(acc_c)

        hard = (sum_hi + (keep + 1.0 - cnt_hi) * tl) / (keep + 1.0)
        full = tot_sum / cnt_valid
        out_ref[0, 0] = jnp.where(keep < cnt_valid, hard, full)


def kernel(pred, target):
    B, C, H, W = pred.shape
    target = target.astype(jnp.int32)
    hblocks = H // _HB
    nblocks = B * hblocks
    total_rows = B * H

    import functools
    body = functools.partial(_ohem_kernel, nblocks=nblocks, total_rows=total_rows)

    out = pl.pallas_call(
        body,
        grid=(nblocks,),
        in_specs=[
            pl.BlockSpec((1, C, _HB, W), lambda i: (i // hblocks, 0, i % hblocks, 0)),
            pl.BlockSpec((1, _HB, W), lambda i: (i // hblocks, i % hblocks, 0)),
        ],
        out_specs=pl.BlockSpec(memory_space=pltpu.SMEM),
        out_shape=jax.ShapeDtypeStruct((1, 1), jnp.float32),
        scratch_shapes=[
            pltpu.VMEM((total_rows, W), jnp.float32),
            pltpu.VMEM((3, _SUB, W), jnp.float32),
        ],
    )(pred, target)
    return out[0, 0]


# speedup vs baseline: 17.2598x; 1.0028x over previous
import jax
import jax.numpy as jnp
from jax import lax
from jax.experimental import pallas as pl
from jax.experimental.pallas import tpu as pltpu

_THRESH = 0.7
_MIN_KEPT = 100000
_IGNORE = 255

_HB = 32
_SUB = 8
_N_ITERS = 45


def _ohem_kernel(pred_ref, tgt_ref, out_ref, loss_buf, acc_ref, nblocks, total_rows):
    i = pl.program_id(0)
    C = pred_ref.shape[1]
    W = pred_ref.shape[3]

    @pl.when(i == 0)
    def _init():
        acc_ref[...] = jnp.zeros_like(acc_ref)

    for r in range(_HB // _SUB):
        rows = pl.ds(r * _SUB, _SUB)
        t = tgt_ref[0, rows, :]
        ms = [
            jnp.maximum(pred_ref[0, 2 * c, rows, :], pred_ref[0, 2 * c + 1, rows, :])
            for c in range(C // 2)
        ]
        if C % 2:
            ms.append(pred_ref[0, C - 1, rows, :])
        while len(ms) > 1:
            nxt = [jnp.maximum(ms[2 * k], ms[2 * k + 1]) for k in range(len(ms) // 2)]
            if len(ms) % 2:
                nxt.append(ms[-1])
            ms = nxt
        m = ms[0]
        ss = [jnp.zeros((_SUB, W), jnp.float32) for _ in range(4)]
        xts = [jnp.zeros((_SUB, W), jnp.float32) for _ in range(2)]
        for c in range(C):
            x = pred_ref[0, c, rows, :]
            ss[c % 4] = ss[c % 4] + jnp.exp(x - m)
            xts[c % 2] = xts[c % 2] + jnp.where(t == c, x, 0.0)
        s = (ss[0] + ss[1]) + (ss[2] + ss[3])
        xt = xts[0] + xts[1]
        valid = t != _IGNORE
        loss = m + jnp.log(s) - xt
        loss = jnp.where(valid, loss, 0.0)
        loss_buf[pl.ds(i * _HB + r * _SUB, _SUB), :] = loss
        acc_ref[0] += jnp.where(valid, 1.0, 0.0)
        acc_ref[1] += loss
        acc_ref[2] = jnp.maximum(acc_ref[2], loss)

    @pl.when(i == nblocks - 1)
    def _finish():
        cnt_valid = jnp.sum(acc_ref[0])
        tot_sum = jnp.sum(acc_ref[1])
        max_loss = jnp.max(acc_ref[2])

        keep = jnp.maximum(
            jnp.float32(_MIN_KEPT), jnp.floor(cnt_valid * jnp.float32(_THRESH))
        )
        keep = jnp.minimum(keep, cnt_valid)

        nchunks = total_rows // _HB

        def count_ge(t):
            def body(j, acc):
                chunk = loss_buf[pl.ds(j * _HB, _HB), :]
                ge = jnp.where(chunk >= t, 1.0, 0.0)
                part = ge[0:_SUB]
                for k in range(1, _HB // _SUB):
                    part = part + ge[k * _SUB:(k + 1) * _SUB]
                return acc + part
            vec = lax.fori_loop(0, nchunks, body, jnp.zeros((_SUB, W), jnp.float32))
            return jnp.sum(vec)

        def bs_body(_, carry):
            tl, th = carry
            tm = 0.5 * (tl + th)
            cm = count_ge(tm)
            take_lo = cm >= keep + 1.0
            return (jnp.where(take_lo, tm, tl), jnp.where(take_lo, th, tm))

        tl0 = jnp.float32(0.0)
        th0 = max_loss * jnp.float32(1.0001) + jnp.float32(1e-6)
        tl, th = lax.fori_loop(0, _N_ITERS, bs_body, (tl0, th0))

        def final_body(j, carry):
            acc_s, acc_c = carry
            chunk = loss_buf[pl.ds(j * _HB, _HB), :]
            ge = jnp.where(chunk >= th, 1.0, 0.0)
            kept = ge * chunk
            ps = kept[0:_SUB]
            pc = ge[0:_SUB]
            for k in range(1, _HB // _SUB):
                ps = ps + kept[k * _SUB:(k + 1) * _SUB]
                pc = pc + ge[k * _SUB:(k + 1) * _SUB]
            return (acc_s + ps, acc_c + pc)

        zero = jnp.zeros((_SUB, W), jnp.float32)
        acc_s, acc_c = lax.fori_loop(0, nchunks, final_body, (zero, zero))
        sum_hi = jnp.sum(acc_s)
        cnt_hi = jnp.sum(acc_c)

        hard = (sum_hi + (keep + 1.0 - cnt_hi) * tl) / (keep + 1.0)
        full = tot_sum / cnt_valid
        out_ref[0, 0] = jnp.where(keep < cnt_valid, hard, full)


def kernel(pred, target):
    B, C, H, W = pred.shape
    target = target.astype(jnp.int32)
    hblocks = H // _HB
    nblocks = B * hblocks
    total_rows = B * H

    import functools
    body = functools.partial(_ohem_kernel, nblocks=nblocks, total_rows=total_rows)

    out = pl.pallas_call(
        body,
        grid=(nblocks,),
        in_specs=[
            pl.BlockSpec((1, C, _HB, W), lambda i: (i // hblocks, 0, i % hblocks, 0)),
            pl.BlockSpec((1, _HB, W), lambda i: (i // hblocks, i % hblocks, 0)),
        ],
        out_specs=pl.BlockSpec(memory_space=pltpu.SMEM),
        out_shape=jax.ShapeDtypeStruct((1, 1), jnp.float32),
        scratch_shapes=[
            pltpu.VMEM((total_rows, W), jnp.float32),
            pltpu.VMEM((3, _SUB, W), jnp.float32),
        ],
    )(pred, target)
    return out[0, 0]


# speedup vs baseline: 24.4046x; 1.4140x over previous
import jax
import jax.numpy as jnp
from jax import lax
from jax.experimental import pallas as pl
from jax.experimental.pallas import tpu as pltpu

_THRESH = 0.7
_MIN_KEPT = 100000
_IGNORE = 255

_HB = 32
_SUB = 8
_N_ITERS = 16


def _ohem_kernel(pred_ref, tgt_ref, out_ref, loss_buf, acc_ref, nblocks, total_rows):
    i = pl.program_id(0)
    C = pred_ref.shape[1]
    W = pred_ref.shape[3]

    @pl.when(i == 0)
    def _init():
        acc_ref[...] = jnp.zeros_like(acc_ref)

    step_cnt = None
    step_sum = None
    step_max = None
    for r in range(_HB // _SUB):
        rows = pl.ds(r * _SUB, _SUB)
        t = tgt_ref[0, rows, :]
        ms = [
            jnp.maximum(pred_ref[0, 2 * c, rows, :], pred_ref[0, 2 * c + 1, rows, :])
            for c in range(C // 2)
        ]
        if C % 2:
            ms.append(pred_ref[0, C - 1, rows, :])
        while len(ms) > 1:
            nxt = [jnp.maximum(ms[2 * k], ms[2 * k + 1]) for k in range(len(ms) // 2)]
            if len(ms) % 2:
                nxt.append(ms[-1])
            ms = nxt
        m = ms[0]
        ss = [jnp.zeros((_SUB, W), jnp.float32) for _ in range(4)]
        xts = [jnp.zeros((_SUB, W), jnp.float32) for _ in range(2)]
        for c in range(C):
            x = pred_ref[0, c, rows, :]
            ss[c % 4] = ss[c % 4] + jnp.exp(x - m)
            xts[c % 2] = xts[c % 2] + jnp.where(t == c, x, 0.0)
        s = (ss[0] + ss[1]) + (ss[2] + ss[3])
        xt = xts[0] + xts[1]
        valid = t != _IGNORE
        loss = m + jnp.log(s) - xt
        loss = jnp.where(valid, loss, 0.0)
        loss_buf[pl.ds(i * _HB + r * _SUB, _SUB), :] = loss
        vf = jnp.where(valid, 1.0, 0.0)
        if step_cnt is None:
            step_cnt, step_sum, step_max = vf, loss, loss
        else:
            step_cnt = step_cnt + vf
            step_sum = step_sum + loss
            step_max = jnp.maximum(step_max, loss)
    acc_ref[0] += step_cnt
    acc_ref[1] += step_sum
    acc_ref[2] = jnp.maximum(acc_ref[2], step_max)

    @pl.when(i == nblocks - 1)
    def _finish():
        cnt_valid = jnp.sum(acc_ref[0])
        tot_sum = jnp.sum(acc_ref[1])
        max_loss = jnp.max(acc_ref[2])

        keep = jnp.maximum(
            jnp.float32(_MIN_KEPT), jnp.floor(cnt_valid * jnp.float32(_THRESH))
        )
        keep = jnp.minimum(keep, cnt_valid)

        nchunks = total_rows // _HB

        UNROLL = 4
        rows_per_iter = UNROLL * _HB

        def count_ge(t):
            def body(j, acc):
                for u in range(UNROLL):
                    chunk = loss_buf[pl.ds(j * rows_per_iter + u * _HB, _HB), :]
                    ge = jnp.where(chunk >= t, 1.0, 0.0)
                    part = ge[0:_SUB]
                    for k in range(1, _HB // _SUB):
                        part = part + ge[k * _SUB:(k + 1) * _SUB]
                    acc = acc + part
                return acc
            vec = lax.fori_loop(
                0, nchunks // UNROLL, body, jnp.zeros((_SUB, W), jnp.float32)
            )
            return jnp.sum(vec)

        target_c = keep + 1.0

        def bs_body(it, carry):
            tl, cl, th, ch = carry
            frac = (cl - target_c) / jnp.maximum(cl - ch, 1.0)
            frac = jnp.clip(frac, 0.08, 0.92)
            tm_interp = tl + frac * (th - tl)
            tm_bisect = 0.5 * (tl + th)
            tm = jnp.where(it % 2 == 0, tm_interp, tm_bisect)
            cm = count_ge(tm)
            take_lo = cm >= target_c
            return (
                jnp.where(take_lo, tm, tl),
                jnp.where(take_lo, cm, cl),
                jnp.where(take_lo, th, tm),
                jnp.where(take_lo, ch, cm),
            )

        tl0 = jnp.float32(0.0)
        cl0 = jnp.float32(total_rows * W)
        th0 = max_loss * jnp.float32(1.0001) + jnp.float32(1e-6)
        tl, _, th, _ = lax.fori_loop(
            0, _N_ITERS, bs_body, (tl0, cl0, th0, jnp.float32(0.0))
        )

        def final_body(j, carry):
            acc_s, acc_c = carry
            for u in range(UNROLL):
                chunk = loss_buf[pl.ds(j * rows_per_iter + u * _HB, _HB), :]
                ge = jnp.where(chunk >= th, 1.0, 0.0)
                kept = ge * chunk
                ps = kept[0:_SUB]
                pc = ge[0:_SUB]
                for k in range(1, _HB // _SUB):
                    ps = ps + kept[k * _SUB:(k + 1) * _SUB]
                    pc = pc + ge[k * _SUB:(k + 1) * _SUB]
                acc_s = acc_s + ps
                acc_c = acc_c + pc
            return (acc_s, acc_c)

        zero = jnp.zeros((_SUB, W), jnp.float32)
        acc_s, acc_c = lax.fori_loop(0, nchunks // UNROLL, final_body, (zero, zero))
        sum_hi = jnp.sum(acc_s)
        cnt_hi = jnp.sum(acc_c)

        hard = (sum_hi + (keep + 1.0 - cnt_hi) * tl) / (keep + 1.0)
        full = tot_sum / cnt_valid
        out_ref[0, 0] = jnp.where(keep < cnt_valid, hard, full)


def kernel(pred, target):
    B, C, H, W = pred.shape
    target = target.astype(jnp.int32)
    hblocks = H // _HB
    nblocks = B * hblocks
    total_rows = B * H

    import functools
    body = functools.partial(_ohem_kernel, nblocks=nblocks, total_rows=total_rows)

    out = pl.pallas_call(
        body,
        grid=(nblocks,),
        in_specs=[
            pl.BlockSpec((1, C, _HB, W), lambda i: (i // hblocks, 0, i % hblocks, 0)),
            pl.BlockSpec((1, _HB, W), lambda i: (i // hblocks, i % hblocks, 0)),
        ],
        out_specs=pl.BlockSpec(memory_space=pltpu.SMEM),
        out_shape=jax.ShapeDtypeStruct((1, 1), jnp.float32),
        scratch_shapes=[
            pltpu.VMEM((total_rows, W), jnp.float32),
            pltpu.VMEM((3, _SUB, W), jnp.float32),
        ],
    )(pred, target)
    return out[0, 0]
